# Initial kernel scaffold; baseline (speedup 1.0000x reference)
import jax
import jax.numpy as jnp
from jax.experimental import pallas as pl


def kernel(features, neighbors, W, b):
    raise NotImplementedError("write your pallas kernel here")


# speedup vs baseline: 1.4119x; 1.4119x over previous
import functools

import jax
import jax.numpy as jnp
from jax import lax
from jax.experimental import pallas as pl
from jax.experimental.pallas import tpu as pltpu
from jax.experimental.pallas import tpu_sc as plsc

N = 10000
DEG = 16
D = 256

NC = 2
NS = 16
NW = NC * NS
NODES_PER_TILE = 320
NPAD = NW * NODES_PER_TILE
CHUNK_NODES = 4
CHUNK_ROWS = CHUNK_NODES * DEG
NCHUNKS = NODES_PER_TILE // CHUNK_NODES
LANES = 16
CB = D // LANES

MM_BLOCK = 1000


def _mlp_kernel(x_ref, wt_ref, b_ref, o_ref):
    o_ref[...] = jnp.maximum(
        jnp.dot(x_ref[...], wt_ref[...], preferred_element_type=jnp.float32)
        + b_ref[...],
        0.0,
    )


def _mlp(features, Wt, b2d):
    return pl.pallas_call(
        _mlp_kernel,
        grid=(N // MM_BLOCK,),
        in_specs=[
            pl.BlockSpec((MM_BLOCK, D), lambda i: (i, 0)),
            pl.BlockSpec((D, D), lambda i: (0, 0)),
            pl.BlockSpec((1, D), lambda i: (0, 0)),
        ],
        out_specs=pl.BlockSpec((MM_BLOCK, D), lambda i: (i, 0)),
        out_shape=jax.ShapeDtypeStruct((N, D), jnp.float32),
    )(features, Wt, b2d)


def _gather_max_body(h_hbm, nbr_hbm, out_hbm, idx_v, bufa, bufb, out_v, sema, semb):
    wid = lax.axis_index("s") * NC + lax.axis_index("c")

    pltpu.sync_copy(nbr_hbm.at[wid], idx_v)

    def start(c, buf, sem):
        pltpu.async_copy(h_hbm.at[idx_v.at[c]], buf, sem)

    def wait(c, buf, sem):
        pltpu.make_async_copy(h_hbm.at[idx_v.at[c]], buf, sem).wait()

    def compute(c, buf):
        def node_body(n, carry):
            for cb in range(CB):
                sl = pl.ds(cb * LANES, LANES)
                acc = buf[n * DEG, sl]
                for r in range(1, DEG):
                    acc = jnp.maximum(acc, buf[n * DEG + r, sl])
                out_v[c * CHUNK_NODES + n, sl] = acc
            return carry

        lax.fori_loop(0, CHUNK_NODES, node_body, 0)

    start(0, bufa, sema)
    start(1, bufb, semb)

    def outer(t, carry):
        c0 = 2 * t
        wait(c0, bufa, sema)
        compute(c0, bufa)

        @pl.when(c0 + 2 < NCHUNKS)
        def _():
            start(c0 + 2, bufa, sema)

        c1 = c0 + 1
        wait(c1, bufb, semb)
        compute(c1, bufb)

        @pl.when(c1 + 2 < NCHUNKS)
        def _():
            start(c1 + 2, bufb, semb)

        return carry

    lax.fori_loop(0, NCHUNKS // 2, outer, 0)

    pltpu.sync_copy(out_v, out_hbm.at[pl.ds(wid * NODES_PER_TILE, NODES_PER_TILE)])


@functools.lru_cache(maxsize=1)
def _build_gather_max():
    mesh = plsc.VectorSubcoreMesh(core_axis_name="c", subcore_axis_name="s")
    return pl.kernel(
        _gather_max_body,
        mesh=mesh,
        out_type=jax.ShapeDtypeStruct((NPAD, D), jnp.float32),
        scratch_types=[
            pltpu.VMEM((NCHUNKS, CHUNK_ROWS), jnp.int32),
            pltpu.VMEM((CHUNK_ROWS, D), jnp.float32),
            pltpu.VMEM((CHUNK_ROWS, D), jnp.float32),
            pltpu.VMEM((NODES_PER_TILE, D), jnp.float32),
            pltpu.SemaphoreType.DMA,
            pltpu.SemaphoreType.DMA,
        ],
    )


def kernel(features, neighbors, W, b):
    h = _mlp(features, W.T, b.reshape(1, D))
    nbr = jnp.concatenate(
        [neighbors, jnp.zeros((NPAD - N, DEG), jnp.int32)], axis=0
    ).reshape(NW, NCHUNKS, CHUNK_ROWS)
    out = _build_gather_max()(h, nbr)
    return out[:N]
